# Initial kernel scaffold; baseline (speedup 1.0000x reference)
import jax
import jax.numpy as jnp
from jax.experimental import pallas as pl


def kernel(bond_types, embedding_weight):
    raise NotImplementedError("write your pallas kernel here")


# speedup vs baseline: 1.0171x; 1.0171x over previous
import functools

import jax
import jax.numpy as jnp
from jax import lax
from jax.experimental import pallas as pl
from jax.experimental.pallas import tpu as pltpu
from jax.experimental.pallas import tpu_sc as plsc

_N = 3_200_000
_D = 16
_NW = 32
_B_PER_W = _N // _NW
_CH = 2_000
_NCH = _B_PER_W // _CH

_mesh = plsc.VectorSubcoreMesh(core_axis_name="c", subcore_axis_name="s")


@functools.partial(
    pl.kernel,
    out_type=jax.ShapeDtypeStruct((_N, _D), jnp.float32),
    mesh=_mesh,
    compiler_params=pltpu.CompilerParams(use_tc_tiling_on_sc=False),
    scratch_types=[
        pltpu.VMEM((_CH,), jnp.int32),
        pltpu.VMEM((_CH, _D), jnp.float32),
        pltpu.SemaphoreType.DMA,
    ],
)
def _emb_lookup(idx_hbm, table_hbm, out_hbm, idx_v, rows_v, sem):
    wid = lax.axis_index("s") * 2 + lax.axis_index("c")
    base = wid * _B_PER_W

    def body(i, carry):
        off = base + i * _CH
        pltpu.sync_copy(idx_hbm.at[pl.ds(off, _CH)], idx_v)
        pltpu.async_copy(table_hbm.at[idx_v], rows_v, sem).wait()
        pltpu.sync_copy(rows_v, out_hbm.at[pl.ds(off, _CH)])
        return carry

    lax.fori_loop(0, _NCH, body, 0)


def kernel(bond_types, embedding_weight):
    return _emb_lookup(bond_types.astype(jnp.int32), embedding_weight)


# speedup vs baseline: 5.7888x; 5.6915x over previous
import functools

import jax
import jax.numpy as jnp
from jax import lax
from jax.experimental import pallas as pl
from jax.experimental.pallas import tpu as pltpu
from jax.experimental.pallas import tpu_sc as plsc

_N = 3_200_000
_D = 16
_NW = 32
_B_PER_W = _N // _NW
_CH = 2_000
_NCH = _B_PER_W // _CH

_mesh = plsc.VectorSubcoreMesh(core_axis_name="c", subcore_axis_name="s")


@functools.partial(
    pl.kernel,
    out_type=jax.ShapeDtypeStruct((_N * _D,), jnp.float32),
    mesh=_mesh,
    compiler_params=pltpu.CompilerParams(use_tc_tiling_on_sc=False,
                                          needs_layout_passes=False),
    scratch_types=[
        pltpu.VMEM((_D * 10,), jnp.float32),
        pltpu.VMEM((_CH,), jnp.int32),
        pltpu.VMEM((_CH * _D,), jnp.float32),
    ],
)
def _emb_lookup(idx_hbm, table_hbm, out_hbm, tbl_v, idx_v, rows_v):
    wid = lax.axis_index("s") * 2 + lax.axis_index("c")
    base = wid * _B_PER_W
    pltpu.sync_copy(table_hbm, tbl_v)
    soff0 = lax.iota(jnp.int32, 16) * _D

    def chunk_body(i, carry):
        off = base + i * _CH
        pltpu.sync_copy(idx_hbm.at[pl.ds(off, _CH)], idx_v)

        def block_body(k, c2):
            idx_vec = idx_v[pl.ds(k * 16, 16)]
            fidx = idx_vec * _D
            sbase = soff0 + k * (16 * _D)
            for j in range(_D):
                col = plsc.load_gather(tbl_v, [fidx + j])
                plsc.store_scatter(rows_v, [sbase + j], col)
            return c2

        lax.fori_loop(0, _CH // 16, block_body, 0)
        pltpu.sync_copy(rows_v, out_hbm.at[pl.ds(off * _D, _CH * _D)])
        return carry

    lax.fori_loop(0, _NCH, chunk_body, 0)


def kernel(bond_types, embedding_weight):
    flat = _emb_lookup(bond_types.astype(jnp.int32),
                       embedding_weight.reshape(-1))
    return flat.reshape(_N, _D)


# speedup vs baseline: 5.7995x; 1.0019x over previous
import functools

import jax
import jax.numpy as jnp
from jax import lax
from jax.experimental import pallas as pl
from jax.experimental.pallas import tpu as pltpu
from jax.experimental.pallas import tpu_sc as plsc

_N = 3_200_000
_D = 16
_NW = 32
_B_PER_W = _N // _NW
_CH = 2_000
_NCH = _B_PER_W // _CH

_mesh = plsc.VectorSubcoreMesh(core_axis_name="c", subcore_axis_name="s")


@functools.partial(
    pl.kernel,
    out_type=jax.ShapeDtypeStruct((_N, _D), jnp.float32),
    mesh=_mesh,
    compiler_params=pltpu.CompilerParams(use_tc_tiling_on_sc=False,
                                          needs_layout_passes=False),
    scratch_types=[
        pltpu.VMEM((_D * 10,), jnp.float32),
        pltpu.VMEM((_CH,), jnp.int32),
        pltpu.VMEM((_CH, _D), jnp.float32),
    ],
)
def _emb_lookup(idx_hbm, table_hbm, out_hbm, tbl_v, idx_v, rows_v):
    wid = lax.axis_index("s") * 2 + lax.axis_index("c")
    base = wid * _B_PER_W
    pltpu.sync_copy(table_hbm, tbl_v)
    iota16 = lax.iota(jnp.int32, 16)

    def chunk_body(i, carry):
        off = base + i * _CH
        pltpu.sync_copy(idx_hbm.at[pl.ds(off, _CH)], idx_v)

        def block_body(k, c2):
            idx_vec = idx_v[pl.ds(k * 16, 16)]
            fidx = idx_vec * _D
            rows = iota16 + k * 16
            for j in range(_D):
                col = plsc.load_gather(tbl_v, [fidx + j])
                plsc.store_scatter(rows_v, [rows, jnp.full((16,), j, jnp.int32)],
                                   col)
            return c2

        lax.fori_loop(0, _CH // 16, block_body, 0)
        pltpu.sync_copy(rows_v, out_hbm.at[pl.ds(off, _CH)])
        return carry

    lax.fori_loop(0, _NCH, chunk_body, 0)


def kernel(bond_types, embedding_weight):
    return _emb_lookup(bond_types.astype(jnp.int32),
                       embedding_weight.reshape(-1))


# speedup vs baseline: 6.7422x; 1.1625x over previous
import functools

import jax
import jax.numpy as jnp
from jax import lax
from jax.experimental import pallas as pl
from jax.experimental.pallas import tpu as pltpu
from jax.experimental.pallas import tpu_sc as plsc

_N = 3_200_000
_D = 16
_NW = 32
_B_PER_W = _N // _NW
_CH = 2_000
_NCH = _B_PER_W // _CH

_mesh = plsc.VectorSubcoreMesh(core_axis_name="c", subcore_axis_name="s")


@functools.partial(
    pl.kernel,
    out_type=jax.ShapeDtypeStruct((_N, _D), jnp.float32),
    mesh=_mesh,
    compiler_params=pltpu.CompilerParams(use_tc_tiling_on_sc=False,
                                          needs_layout_passes=False),
    scratch_types=[
        pltpu.VMEM((_D * 10,), jnp.float32),
        pltpu.VMEM((_CH,), jnp.int32),
        pltpu.VMEM((_CH, _D), jnp.float32),
    ],
)
def _emb_lookup(idx_hbm, table_hbm, out_hbm, tbl_v, idx_v, rows_v):
    wid = lax.axis_index("s") * 2 + lax.axis_index("c")
    base = wid * _B_PER_W
    pltpu.sync_copy(table_hbm, tbl_v)
    iota16 = lax.iota(jnp.int32, 16)

    def chunk_body(i, carry):
        off = base + i * _CH
        pltpu.sync_copy(idx_hbm.at[pl.ds(off, _CH)], idx_v)

        @plsc.parallel_loop(0, _CH // 16, unroll=4)
        def block_body(k):
            idx_vec = idx_v[pl.ds(k * 16, 16)]
            fidx = idx_vec * _D
            rows = iota16 + k * 16
            for j in range(_D):
                col = plsc.load_gather(tbl_v, [fidx + j])
                plsc.store_scatter(rows_v, [rows, jnp.full((16,), j, jnp.int32)],
                                   col)
        pltpu.sync_copy(rows_v, out_hbm.at[pl.ds(off, _CH)])
        return carry

    lax.fori_loop(0, _NCH, chunk_body, 0)


def kernel(bond_types, embedding_weight):
    return _emb_lookup(bond_types.astype(jnp.int32),
                       embedding_weight.reshape(-1))


# speedup vs baseline: 6.8322x; 1.0134x over previous
import functools

import jax
import jax.numpy as jnp
from jax import lax
from jax.experimental import pallas as pl
from jax.experimental.pallas import tpu as pltpu
from jax.experimental.pallas import tpu_sc as plsc

_N = 3_200_000
_D = 16
_NW = 32
_B_PER_W = _N // _NW
_CH = 2_000
_NCH = _B_PER_W // _CH

_mesh = plsc.VectorSubcoreMesh(core_axis_name="c", subcore_axis_name="s")


@functools.partial(
    pl.kernel,
    out_type=jax.ShapeDtypeStruct((_N * _D,), jnp.float32),
    mesh=_mesh,
    compiler_params=pltpu.CompilerParams(use_tc_tiling_on_sc=False,
                                         needs_layout_passes=False),
    scratch_types=[
        pltpu.VMEM((_D * 10,), jnp.float32),
        [pltpu.VMEM((_CH,), jnp.int32)] * 2,
        [pltpu.VMEM((_CH * _D,), jnp.float32)] * 2,
        [pltpu.SemaphoreType.DMA] * 2,
        [pltpu.SemaphoreType.DMA] * 2,
    ],
)
def _emb_lookup(idx_hbm, table_hbm, out_hbm, tbl_v, idx_v, rows_v, isem, osem):
    wid = lax.axis_index("s") * 2 + lax.axis_index("c")
    base = wid * _B_PER_W
    pltpu.sync_copy(table_hbm, tbl_v)
    iota16t = lax.iota(jnp.int32, 16) * _D

    def idx_copy(c, b):
        return pltpu.make_async_copy(
            idx_hbm.at[pl.ds(base + c * _CH, _CH)], idx_v[b], isem[b])

    def out_copy(c, b):
        return pltpu.make_async_copy(
            rows_v[b], out_hbm.at[pl.ds((base + c * _CH) * _D, _CH * _D)],
            osem[b])

    def expand(b):
        @plsc.parallel_loop(0, _CH // 16, unroll=5)
        def blk(k):
            idx_vec = idx_v[b][pl.ds(k * 16, 16)]
            fidx = idx_vec * _D
            sbase = iota16t + k * (16 * _D)
            for j in range(_D):
                col = plsc.load_gather(tbl_v, [fidx + j])
                plsc.store_scatter(rows_v[b], [sbase + j], col)

    idx_copy(0, 0).start()
    idx_copy(1, 1).start()
    for b in range(2):
        idx_copy(b, b).wait()
        expand(b)
        out_copy(b, b).start()
        idx_copy(b + 2, b).start()

    def pair_body(p, carry):
        c0 = 2 * p
        for b in range(2):
            c = c0 + b
            idx_copy(c, b).wait()
            out_copy(c - 2, b).wait()
            expand(b)
            out_copy(c, b).start()
            idx_copy(c + 2, b).start()
        return carry

    lax.fori_loop(1, _NCH // 2 - 1, pair_body, 0)

    for b in range(2):
        c = _NCH - 2 + b
        idx_copy(c, b).wait()
        out_copy(c - 2, b).wait()
        expand(b)
        out_copy(c, b).start()
    out_copy(_NCH - 2, 0).wait()
    out_copy(_NCH - 1, 1).wait()


def kernel(bond_types, embedding_weight):
    flat = _emb_lookup(bond_types.astype(jnp.int32),
                       embedding_weight.reshape(-1))
    return flat.reshape(_N, _D)
